# Initial kernel scaffold; baseline (speedup 1.0000x reference)
import jax
import jax.numpy as jnp
from jax.experimental import pallas as pl


def kernel(indexes, index2vec_weight):
    raise NotImplementedError("write your pallas kernel here")


# speedup vs baseline: 1.6372x; 1.6372x over previous
import jax
import jax.numpy as jnp
from jax import lax
from jax.experimental import pallas as pl
from jax.experimental.pallas import tpu as pltpu
from jax.experimental.pallas import tpu_sc as plsc

D = 32
LANE = 128
K = 8
NC, NS = 2, 16
NW = NC * NS


def _gather_body(idx_hbm, tbl_hbm, out_hbm, idx_v, rows_v, sem):
    wid = lax.axis_index("s") * NC + lax.axis_index("c")
    rows_per_w = idx_hbm.shape[0] // NW
    row0 = wid * rows_per_w
    nch = rows_per_w // K

    def chunk(i, carry):
        base = row0 + i * K
        pltpu.sync_copy(idx_hbm.at[pl.ds(base, K)], idx_v)
        cps = [
            pltpu.async_copy(tbl_hbm.at[idx_v.at[j]], rows_v.at[j], sem)
            for j in range(K)
        ]
        for c in cps:
            c.wait()
        pltpu.sync_copy(rows_v, out_hbm.at[pl.ds(base, K)])
        return carry

    lax.fori_loop(0, nch, chunk, 0)


def kernel(indexes, index2vec_weight):
    n = indexes.shape[0]
    idx2 = indexes.reshape(n // LANE, LANE)
    mesh = plsc.VectorSubcoreMesh(core_axis_name="c", subcore_axis_name="s")
    f = pl.kernel(
        _gather_body,
        out_type=jax.ShapeDtypeStruct((n // LANE, LANE, D), jnp.float32),
        mesh=mesh,
        scratch_types=[
            pltpu.VMEM((K, LANE), jnp.int32),
            pltpu.VMEM((K, LANE, D), jnp.float32),
            pltpu.SemaphoreType.DMA,
        ],
        compiler_params=pltpu.CompilerParams(use_tc_tiling_on_sc=False),
    )
    out = f(idx2, index2vec_weight)
    return out.reshape(n, D)


# speedup vs baseline: 1.7153x; 1.0477x over previous
import jax
import jax.numpy as jnp
from jax import lax
from jax.experimental import pallas as pl
from jax.experimental.pallas import tpu as pltpu
from jax.experimental.pallas import tpu_sc as plsc

D = 32
LANE = 128
K = 8
NIB = 4
NRB = 2
NC, NS = 2, 16
NW = NC * NS


def _gather_body(idx_hbm, tbl_hbm, out_hbm, idx_v, rows_v, isem, gsem, osem):
    wid = lax.axis_index("s") * NC + lax.axis_index("c")
    rows_per_w = idx_hbm.shape[0] // NW
    row0 = wid * rows_per_w
    nch = rows_per_w // K

    def fire_idx(g, ib):
        pltpu.async_copy(
            idx_hbm.at[pl.ds(row0 + g * K, K)], idx_v.at[ib], isem.at[ib]
        )

    def wait_idx(ib):
        pltpu.make_async_copy(
            idx_hbm.at[pl.ds(row0, K)], idx_v.at[ib], isem.at[ib]
        ).wait()

    def wait_store(rb):
        pltpu.make_async_copy(
            rows_v.at[rb], out_hbm.at[pl.ds(row0, K)], osem.at[rb]
        ).wait()

    for g in range(NIB - 1):
        fire_idx(g, g)

    def outer(i, carry):
        for u in range(NIB):
            g = NIB * i + u
            ib = u
            rb = u % NRB
            wait_idx(ib)

            @pl.when(g >= NRB)
            def _():
                wait_store(rb)

            gth = [
                pltpu.async_copy(
                    tbl_hbm.at[idx_v.at[ib, j]], rows_v.at[rb, j], gsem
                )
                for j in range(K)
            ]

            @pl.when(g + NIB - 1 < nch)
            def _():
                fire_idx(g + NIB - 1, (u + NIB - 1) % NIB)

            for c in gth:
                c.wait()
            pltpu.async_copy(
                rows_v.at[rb], out_hbm.at[pl.ds(row0 + g * K, K)], osem.at[rb]
            )
        return carry

    lax.fori_loop(0, nch // NIB, outer, 0)
    for rb in range(NRB):
        wait_store(rb)


def kernel(indexes, index2vec_weight):
    n = indexes.shape[0]
    nrows = n // LANE
    assert nrows % (NW * K * NIB) == 0
    idx2 = indexes.reshape(nrows, LANE)
    mesh = plsc.VectorSubcoreMesh(core_axis_name="c", subcore_axis_name="s")
    f = pl.kernel(
        _gather_body,
        out_type=jax.ShapeDtypeStruct((nrows, LANE, D), jnp.float32),
        mesh=mesh,
        scratch_types=[
            pltpu.VMEM((NIB, K, LANE), jnp.int32),
            pltpu.VMEM((NRB, K, LANE, D), jnp.float32),
            pltpu.SemaphoreType.DMA((NIB,)),
            pltpu.SemaphoreType.DMA,
            pltpu.SemaphoreType.DMA((NRB,)),
        ],
        compiler_params=pltpu.CompilerParams(use_tc_tiling_on_sc=False),
    )
    out = f(idx2, index2vec_weight)
    return out.reshape(n, D)


# speedup vs baseline: 1.7204x; 1.0029x over previous
import jax
import jax.numpy as jnp
from jax import lax
from jax.experimental import pallas as pl
from jax.experimental.pallas import tpu as pltpu
from jax.experimental.pallas import tpu_sc as plsc

D = 32
LANE = 128
K = 8
NIB = 4
NRB = 2
NC, NS = 2, 16
NW = NC * NS


def _gather_body(idx_hbm, tbl_hbm, out_hbm, idx_v, rows_v, isem, gsem, osem):
    wid = lax.axis_index("s") * NC + lax.axis_index("c")
    rows_per_w = idx_hbm.shape[0] // NW
    row0 = wid * rows_per_w
    nch = rows_per_w // K

    def fire_idx(g, ib):
        pltpu.async_copy(
            idx_hbm.at[pl.ds(row0 + g * K, K)], idx_v.at[ib], isem.at[ib]
        )

    def wait_idx(ib):
        pltpu.make_async_copy(
            idx_hbm.at[pl.ds(row0, K)], idx_v.at[ib], isem.at[ib]
        ).wait()

    def fire_gathers(ib, rb):
        for j in range(K):
            pltpu.async_copy(
                tbl_hbm.at[idx_v.at[ib, j]], rows_v.at[rb, j], gsem.at[rb]
            )

    def drain_gathers(rb):
        for j in range(K):
            pltpu.make_async_copy(
                tbl_hbm.at[idx_v.at[0, 0]], rows_v.at[rb, j], gsem.at[rb]
            ).wait()

    def fire_store(g, rb):
        pltpu.async_copy(
            rows_v.at[rb], out_hbm.at[pl.ds(row0 + g * K, K)], osem.at[rb]
        )

    def wait_store(rb):
        pltpu.make_async_copy(
            rows_v.at[rb], out_hbm.at[pl.ds(row0, K)], osem.at[rb]
        ).wait()

    for g in range(NIB - 1):
        fire_idx(g, g)

    def outer(i, carry):
        for u in range(NIB):
            g = NIB * i + u
            ib = u
            rb = u % NRB

            @pl.when(g >= NRB)
            def _():
                wait_store(rb)

            wait_idx(ib)
            fire_gathers(ib, rb)

            @pl.when(g >= 1)
            def _():
                drain_gathers(1 - rb)
                fire_store(g - 1, 1 - rb)

            @pl.when(g + NIB - 1 < nch)
            def _():
                fire_idx(g + NIB - 1, (u + NIB - 1) % NIB)
        return carry

    lax.fori_loop(0, nch // NIB, outer, 0)
    last_rb = (nch - 1) % NRB
    drain_gathers(last_rb)
    fire_store(nch - 1, last_rb)
    for rb in range(NRB):
        wait_store(rb)


def kernel(indexes, index2vec_weight):
    n = indexes.shape[0]
    nrows = n // LANE
    assert nrows % (NW * K * NIB) == 0
    idx2 = indexes.reshape(nrows, LANE)
    mesh = plsc.VectorSubcoreMesh(core_axis_name="c", subcore_axis_name="s")
    f = pl.kernel(
        _gather_body,
        out_type=jax.ShapeDtypeStruct((nrows, LANE, D), jnp.float32),
        mesh=mesh,
        scratch_types=[
            pltpu.VMEM((NIB, K, LANE), jnp.int32),
            pltpu.VMEM((NRB, K, LANE, D), jnp.float32),
            pltpu.SemaphoreType.DMA((NIB,)),
            pltpu.SemaphoreType.DMA((NRB,)),
            pltpu.SemaphoreType.DMA((NRB,)),
        ],
        compiler_params=pltpu.CompilerParams(use_tc_tiling_on_sc=False),
    )
    out = f(idx2, index2vec_weight)
    return out.reshape(n, D)
